# baseline (speedup 1.0000x reference)
import jax
import jax.numpy as jnp
from jax.experimental import pallas as pl


def _zero_kernel(o_ref):
    o_ref[...] = jnp.zeros_like(o_ref)


def kernel(x, x_new):
    del x, x_new
    return pl.pallas_call(
        _zero_kernel,
        out_shape=jax.ShapeDtypeStruct((1,), jnp.float32),
    )()
